# Initial kernel scaffold; baseline (speedup 1.0000x reference)
import jax
import jax.numpy as jnp
from jax.experimental import pallas as pl


def kernel(xb):
    raise NotImplementedError("write your pallas kernel here")


# speedup vs baseline: 1.5632x; 1.5632x over previous
import functools

import jax
import jax.numpy as jnp
from jax.experimental import pallas as pl

_MASK_RATIO = 0.15


def _mask_body(noise_r_ref, noise_c_ref, x_ref, xm_ref, mask_ref, *,
               len_keep, tl, nvars):
    t = pl.program_id(1)
    L = noise_r_ref.shape[2]
    n_row = noise_r_ref[0, :, :]
    n_col = noise_c_ref[0, pl.ds(t * tl, tl), :]
    lt = n_row < n_col
    eq = n_row == n_col
    m_idx = jax.lax.broadcasted_iota(jnp.int32, (tl, L), 1)
    l_idx = t * tl + jax.lax.broadcasted_iota(jnp.int32, (tl, L), 0)
    cmp = jnp.logical_or(lt, jnp.logical_and(eq, m_idx < l_idx))
    rank = jnp.sum(cmp.astype(jnp.int32), axis=1, keepdims=True)
    keep = (rank < len_keep).astype(jnp.float32)
    xm_ref[...] = x_ref[...] * keep[None, :, :]
    mask_ref[...] = jnp.broadcast_to((1.0 - keep)[None, :, :], (1, tl, nvars))


@functools.partial(jax.jit, static_argnames=("tl",))
def _run(xb, tl=256):
    bs, L, nvars, D = xb.shape
    len_keep = int(L * (1 - _MASK_RATIO))
    noise = jax.random.uniform(jax.random.key(42), (bs, L), dtype=jnp.float32)
    noise_r = noise.reshape(bs, 1, L)
    noise_c = noise.reshape(bs, L, 1)
    F = nvars * D
    xq = xb.reshape(bs, L, F)
    grid = (bs, L // tl)
    xm, mask = pl.pallas_call(
        functools.partial(_mask_body, len_keep=len_keep, tl=tl, nvars=nvars),
        grid=grid,
        in_specs=[
            pl.BlockSpec((1, 1, L), lambda b, t: (b, 0, 0)),
            pl.BlockSpec((1, L, 1), lambda b, t: (b, 0, 0)),
            pl.BlockSpec((1, tl, F), lambda b, t: (b, t, 0)),
        ],
        out_specs=[
            pl.BlockSpec((1, tl, F), lambda b, t: (b, t, 0)),
            pl.BlockSpec((1, tl, nvars), lambda b, t: (b, t, 0)),
        ],
        out_shape=[
            jax.ShapeDtypeStruct((bs, L, F), xb.dtype),
            jax.ShapeDtypeStruct((bs, L, nvars), jnp.float32),
        ],
    )(noise_r, noise_c, xq)
    return xm.reshape(bs, L, nvars, D), mask


def kernel(xb):
    return _run(xb)


# speedup vs baseline: 1.5651x; 1.0012x over previous
import functools

import jax
import jax.numpy as jnp
from jax.experimental import pallas as pl
from jax.experimental.pallas import tpu as pltpu

_MASK_RATIO = 0.15


def _mask_body(noise_r_ref, noise_c_ref, x_ref, xm_ref, mask_ref, *,
               len_keep, tl, nvars):
    t = pl.program_id(1)
    L = noise_r_ref.shape[2]
    n_row = noise_r_ref[0, :, :]
    n_col = noise_c_ref[0, pl.ds(t * tl, tl), :]
    lt = n_row < n_col
    eq = n_row == n_col
    m_idx = jax.lax.broadcasted_iota(jnp.int32, (tl, L), 1)
    l_idx = t * tl + jax.lax.broadcasted_iota(jnp.int32, (tl, L), 0)
    cmp = jnp.logical_or(lt, jnp.logical_and(eq, m_idx < l_idx))
    rank = jnp.sum(cmp.astype(jnp.int32), axis=1, keepdims=True)
    keep = (rank < len_keep).astype(jnp.float32)
    xm_ref[...] = x_ref[...] * keep[None, :, :]
    mask_ref[...] = jnp.broadcast_to((1.0 - keep)[None, :, :], (1, tl, nvars))


@functools.partial(jax.jit, static_argnames=("tl",))
def _run(xb, tl=256):
    bs, L, nvars, D = xb.shape
    len_keep = int(L * (1 - _MASK_RATIO))
    noise = jax.random.uniform(jax.random.key(42), (bs, L), dtype=jnp.float32)
    noise_r = noise.reshape(bs, 1, L)
    noise_c = noise.reshape(bs, L, 1)
    F = nvars * D
    xq = xb.reshape(bs, L, F)
    grid = (bs, L // tl)
    xm, mask = pl.pallas_call(
        functools.partial(_mask_body, len_keep=len_keep, tl=tl, nvars=nvars),
        grid=grid,
        in_specs=[
            pl.BlockSpec((1, 1, L), lambda b, t: (b, 0, 0)),
            pl.BlockSpec((1, L, 1), lambda b, t: (b, 0, 0)),
            pl.BlockSpec((1, tl, F), lambda b, t: (b, t, 0)),
        ],
        out_specs=[
            pl.BlockSpec((1, tl, F), lambda b, t: (b, t, 0)),
            pl.BlockSpec((1, tl, nvars), lambda b, t: (b, t, 0)),
        ],
        out_shape=[
            jax.ShapeDtypeStruct((bs, L, F), xb.dtype),
            jax.ShapeDtypeStruct((bs, L, nvars), jnp.float32),
        ],
        compiler_params=pltpu.CompilerParams(
            dimension_semantics=("parallel", "parallel"),
        ),
    )(noise_r, noise_c, xq)
    return xm.reshape(bs, L, nvars, D), mask


def kernel(xb):
    return _run(xb)


# speedup vs baseline: 1.5776x; 1.0080x over previous
import functools

import jax
import jax.numpy as jnp
from jax.experimental import pallas as pl
from jax.experimental.pallas import tpu as pltpu

_MASK_RATIO = 0.15


def _mask_body(noise_r_ref, noise_c_ref, x_ref, xm_ref, mask_ref, *,
               len_keep, tl, nvars):
    t = pl.program_id(1)
    L = noise_r_ref.shape[2]
    n_row = noise_r_ref[0, :, :]
    n_col = noise_c_ref[0, pl.ds(t * tl, tl), :]
    lt = n_row < n_col
    eq = n_row == n_col
    m_idx = jax.lax.broadcasted_iota(jnp.int32, (tl, L), 1)
    l_idx = t * tl + jax.lax.broadcasted_iota(jnp.int32, (tl, L), 0)
    cmp = jnp.logical_or(lt, jnp.logical_and(eq, m_idx < l_idx))
    rank = jnp.sum(cmp.astype(jnp.int32), axis=1, keepdims=True)
    keep = (rank < len_keep).astype(jnp.float32)
    xm_ref[...] = x_ref[...] * keep[None, :, :]
    mask_ref[...] = jnp.broadcast_to((1.0 - keep)[None, :, :], (1, tl, nvars))


@functools.partial(jax.jit, static_argnames=("tl",))
def _run(xb, tl=512):
    bs, L, nvars, D = xb.shape
    len_keep = int(L * (1 - _MASK_RATIO))
    noise = jax.random.uniform(jax.random.key(42), (bs, L), dtype=jnp.float32)
    noise_r = noise.reshape(bs, 1, L)
    noise_c = noise.reshape(bs, L, 1)
    F = nvars * D
    xq = xb.reshape(bs, L, F)
    grid = (bs, L // tl)
    xm, mask = pl.pallas_call(
        functools.partial(_mask_body, len_keep=len_keep, tl=tl, nvars=nvars),
        grid=grid,
        in_specs=[
            pl.BlockSpec((1, 1, L), lambda b, t: (b, 0, 0)),
            pl.BlockSpec((1, L, 1), lambda b, t: (b, 0, 0)),
            pl.BlockSpec((1, tl, F), lambda b, t: (b, t, 0)),
        ],
        out_specs=[
            pl.BlockSpec((1, tl, F), lambda b, t: (b, t, 0)),
            pl.BlockSpec((1, tl, nvars), lambda b, t: (b, t, 0)),
        ],
        out_shape=[
            jax.ShapeDtypeStruct((bs, L, F), xb.dtype),
            jax.ShapeDtypeStruct((bs, L, nvars), jnp.float32),
        ],
        compiler_params=pltpu.CompilerParams(
            dimension_semantics=("parallel", "parallel"),
        ),
    )(noise_r, noise_c, xq)
    return xm.reshape(bs, L, nvars, D), mask


def kernel(xb):
    return _run(xb)


# speedup vs baseline: 2.6437x; 1.6758x over previous
import functools

import jax
import jax.numpy as jnp
from jax.experimental import pallas as pl
from jax.experimental.pallas import tpu as pltpu

_MASK_RATIO = 0.15


def _mask_body(noise_r_ref, noise_c_ref, x_ref, xm_ref, mask_ref, *,
               len_keep, tl, nvars):
    t = pl.program_id(1)
    L = noise_r_ref.shape[2]
    n_row = noise_r_ref[0, :, :]
    n_col = noise_c_ref[0, pl.ds(t * tl, tl), :]
    lt = n_row < n_col
    eq = n_row == n_col
    m_idx = jax.lax.broadcasted_iota(jnp.int32, (tl, L), 1)
    l_idx = t * tl + jax.lax.broadcasted_iota(jnp.int32, (tl, L), 0)
    cmp = jnp.logical_or(lt, jnp.logical_and(eq, m_idx < l_idx))
    rank = jnp.sum(cmp.astype(jnp.int32), axis=1, keepdims=True)
    keep = (rank < len_keep).astype(jnp.float32)
    xm_ref[...] = x_ref[...] * keep[None, :, :, None]
    mask_ref[...] = jnp.broadcast_to((1.0 - keep)[None, :, :], (1, tl, nvars))


@functools.partial(jax.jit, static_argnames=("tl",))
def _run(xb, tl=512):
    bs, L, nvars, D = xb.shape
    len_keep = int(L * (1 - _MASK_RATIO))
    noise = jax.random.uniform(jax.random.key(42), (bs, L), dtype=jnp.float32)
    noise_r = noise.reshape(bs, 1, L)
    noise_c = noise.reshape(bs, L, 1)
    grid = (bs, L // tl)
    xm, mask = pl.pallas_call(
        functools.partial(_mask_body, len_keep=len_keep, tl=tl, nvars=nvars),
        grid=grid,
        in_specs=[
            pl.BlockSpec((1, 1, L), lambda b, t: (b, 0, 0)),
            pl.BlockSpec((1, L, 1), lambda b, t: (b, 0, 0)),
            pl.BlockSpec((1, tl, nvars, D), lambda b, t: (b, t, 0, 0)),
        ],
        out_specs=[
            pl.BlockSpec((1, tl, nvars, D), lambda b, t: (b, t, 0, 0)),
            pl.BlockSpec((1, tl, nvars), lambda b, t: (b, t, 0)),
        ],
        out_shape=[
            jax.ShapeDtypeStruct((bs, L, nvars, D), xb.dtype),
            jax.ShapeDtypeStruct((bs, L, nvars), jnp.float32),
        ],
        compiler_params=pltpu.CompilerParams(
            dimension_semantics=("parallel", "parallel"),
        ),
    )(noise_r, noise_c, xb)
    return xm, mask


def kernel(xb):
    return _run(xb)
